# Initial kernel scaffold; baseline (speedup 1.0000x reference)
import jax
import jax.numpy as jnp
from jax.experimental import pallas as pl


def kernel(hiddens, edge_index, y, W, b):
    raise NotImplementedError("write your pallas kernel here")


# speedup vs baseline: 46.1403x; 46.1403x over previous
import functools

import jax
import jax.numpy as jnp
from jax import lax
from jax.experimental import pallas as pl
from jax.experimental.pallas import tpu as pltpu
from jax.experimental.pallas import tpu_sc as plsc

NC = 2
NS = 16
NW = NC * NS


def _sc_mesh():
    return plsc.VectorSubcoreMesh(core_axis_name="c", subcore_axis_name="s")


def _make_deg_kernel(n, nch, ch):
    @functools.partial(
        pl.kernel,
        out_type=jax.ShapeDtypeStruct((NC, n), jnp.float32),
        mesh=_sc_mesh(),
        scratch_types=[
            pltpu.VMEM((nch, ch), jnp.int32),
            pltpu.VMEM((ch,), jnp.float32),
            pltpu.VMEM_SHARED((n,), jnp.float32),
        ],
    )
    def deg_kernel(dst_hbm, ones_hbm, zeros_hbm, out_hbm, idx_v, ones_v, acc_sh):
        c = lax.axis_index("c")
        s = lax.axis_index("s")
        wid = c * NS + s
        pltpu.sync_copy(dst_hbm.at[wid], idx_v)
        pltpu.sync_copy(ones_hbm, ones_v)

        @pl.when(s == 0)
        def _():
            pltpu.sync_copy(zeros_hbm, acc_sh)

        plsc.subcore_barrier()

        def body(j, carry):
            pltpu.sync_copy(ones_v, acc_sh.at[idx_v.at[j]], add=True)
            return carry

        lax.fori_loop(0, nch, body, 0)
        plsc.subcore_barrier()

        @pl.when(s == 0)
        def _():
            pltpu.sync_copy(acc_sh, out_hbm.at[c])

    return deg_kernel


def _make_agg_kernel(n, ncls, nch, ch):
    @functools.partial(
        pl.kernel,
        out_type=jax.ShapeDtypeStruct((NC, n, ncls), jnp.float32),
        mesh=_sc_mesh(),
        compiler_params=pltpu.CompilerParams(use_tc_tiling_on_sc=False),
        scratch_types=[
            pltpu.VMEM((nch, ch), jnp.int32),
            pltpu.VMEM((nch, ch), jnp.int32),
            pltpu.VMEM((ch, ncls), jnp.float32),
            pltpu.VMEM_SHARED((n, ncls), jnp.float32),
            pltpu.SemaphoreType.DMA,
        ],
    )
    def agg_kernel(src_hbm, dst_hbm, z_hbm, zeros_hbm, out_hbm,
                   src_v, dst_v, rows_v, acc_sh, sem):
        c = lax.axis_index("c")
        s = lax.axis_index("s")
        wid = c * NS + s
        pltpu.sync_copy(src_hbm.at[wid], src_v)
        pltpu.sync_copy(dst_hbm.at[wid], dst_v)

        @pl.when(s == 0)
        def _():
            pltpu.sync_copy(zeros_hbm, acc_sh)

        plsc.subcore_barrier()

        def body(j, carry):
            pltpu.async_copy(z_hbm.at[src_v.at[j]], rows_v, sem).wait()
            pltpu.sync_copy(rows_v, acc_sh.at[dst_v.at[j]], add=True)
            return carry

        lax.fori_loop(0, nch, body, 0)
        plsc.subcore_barrier()

        @pl.when(s == 0)
        def _():
            pltpu.sync_copy(acc_sh, out_hbm.at[c])

    return agg_kernel


def _xw_body(h_ref, w_ref, deg_ref, z_ref, dis_ref):
    deg = deg_ref[0] + deg_ref[1] + 1.0
    dis = lax.rsqrt(deg)
    xw = jnp.dot(h_ref[...], w_ref[...], preferred_element_type=jnp.float32)
    z_ref[...] = dis * xw
    dis_ref[...] = dis


def _loss_body(agg_ref, z_ref, dis_ref, y_ref, b_ref, out_ref):
    n, ncls = z_ref.shape
    dis = dis_ref[...]
    out = dis * (agg_ref[0] + agg_ref[1] + z_ref[...]) + b_ref[...]
    m = jnp.max(out, axis=1, keepdims=True)
    lse = m + jnp.log(jnp.sum(jnp.exp(out - m), axis=1, keepdims=True))
    cls_ids = lax.broadcasted_iota(jnp.int32, (n, ncls), 1)
    picked = jnp.sum(jnp.where(cls_ids == y_ref[...], out, 0.0),
                     axis=1, keepdims=True)
    out_ref[...] = jnp.sum(lse - picked, axis=0, keepdims=True) / n


def kernel(hiddens, edge_index, y, W, b):
    n, hid = hiddens.shape
    ncls = W.shape[1]
    e = edge_index.shape[1]
    ch = 125
    nch = e // (NW * ch)
    assert NW * nch * ch == e

    src = edge_index[0].reshape(NW, nch, ch)
    dst = edge_index[1].reshape(NW, nch, ch)
    ones_ch = jnp.ones((ch,), jnp.float32)
    zeros_n = jnp.zeros((n,), jnp.float32)
    zeros_n8 = jnp.zeros((n, ncls), jnp.float32)

    deg2 = _make_deg_kernel(n, nch, ch)(dst, ones_ch, zeros_n)

    z, dis = pl.pallas_call(
        _xw_body,
        out_shape=[
            jax.ShapeDtypeStruct((n, ncls), jnp.float32),
            jax.ShapeDtypeStruct((n, 1), jnp.float32),
        ],
    )(hiddens, W, deg2.reshape(NC, n, 1))

    agg2 = _make_agg_kernel(n, ncls, nch, ch)(src, dst, z, zeros_n8)

    loss = pl.pallas_call(
        _loss_body,
        out_shape=jax.ShapeDtypeStruct((1, 1), jnp.float32),
    )(agg2, z, dis, y.reshape(n, 1), b.reshape(1, ncls))

    return loss.reshape(())


# speedup vs baseline: 66.0625x; 1.4318x over previous
import functools

import jax
import jax.numpy as jnp
from jax import lax
from jax.experimental import pallas as pl
from jax.experimental.pallas import tpu as pltpu
from jax.experimental.pallas import tpu_sc as plsc

NC = 2
NS = 16
NW = NC * NS


def _sc_mesh():
    return plsc.VectorSubcoreMesh(core_axis_name="c", subcore_axis_name="s")


def _make_deg_kernel(n, nch, ch):
    @functools.partial(
        pl.kernel,
        out_type=jax.ShapeDtypeStruct((NC, n), jnp.float32),
        mesh=_sc_mesh(),
        scratch_types=[
            pltpu.VMEM((nch, ch), jnp.int32),
            pltpu.VMEM((ch,), jnp.float32),
            pltpu.VMEM_SHARED((n,), jnp.float32),
            pltpu.SemaphoreType.DMA,
        ],
    )
    def deg_kernel(dst_hbm, ones_hbm, zeros_hbm, out_hbm, idx_v, ones_v,
                   acc_sh, sem):
        c = lax.axis_index("c")
        s = lax.axis_index("s")
        wid = c * NS + s
        pltpu.sync_copy(dst_hbm.at[wid], idx_v)
        pltpu.sync_copy(ones_hbm, ones_v)

        @pl.when(s == 0)
        def _():
            pltpu.sync_copy(zeros_hbm, acc_sh)

        plsc.subcore_barrier()

        k = 8
        def body(g, carry):
            for b in range(k):
                pltpu.async_copy(ones_v, acc_sh.at[idx_v.at[g + b]], sem,
                                 add=True)
            for b in range(k):
                pltpu.make_async_copy(ones_v, acc_sh.at[idx_v.at[g + b]],
                                      sem).wait()
            return carry

        lax.fori_loop(0, nch // k, lambda g, c: body(g * k, c), 0,
                      unroll=False)
        plsc.subcore_barrier()

        @pl.when(s == 0)
        def _():
            pltpu.sync_copy(acc_sh, out_hbm.at[c])

    return deg_kernel


def _make_agg_kernel(n, ncls, nch, ch):
    @functools.partial(
        pl.kernel,
        out_type=jax.ShapeDtypeStruct((NC, n, ncls), jnp.float32),
        mesh=_sc_mesh(),
        compiler_params=pltpu.CompilerParams(use_tc_tiling_on_sc=False),
        scratch_types=[
            pltpu.VMEM((nch, ch), jnp.int32),
            pltpu.VMEM((nch, ch), jnp.int32),
            pltpu.VMEM((8, ch, ncls), jnp.float32),
            pltpu.VMEM_SHARED((n, ncls), jnp.float32),
            pltpu.SemaphoreType.DMA,
            pltpu.SemaphoreType.DMA,
        ],
    )
    def agg_kernel(src_hbm, dst_hbm, z_hbm, zeros_hbm, out_hbm,
                   src_v, dst_v, rows_v, acc_sh, gsem, ssem):
        c = lax.axis_index("c")
        s = lax.axis_index("s")
        wid = c * NS + s
        pltpu.sync_copy(src_hbm.at[wid], src_v)
        pltpu.sync_copy(dst_hbm.at[wid], dst_v)

        @pl.when(s == 0)
        def _():
            pltpu.sync_copy(zeros_hbm, acc_sh)

        plsc.subcore_barrier()

        k = 8
        def body(g, carry):
            for b in range(k):
                pltpu.async_copy(z_hbm.at[src_v.at[g + b]], rows_v.at[b],
                                 gsem)
            for b in range(k):
                pltpu.make_async_copy(z_hbm.at[src_v.at[g + b]],
                                      rows_v.at[b], gsem).wait()
            for b in range(k):
                pltpu.async_copy(rows_v.at[b], acc_sh.at[dst_v.at[g + b]],
                                 ssem, add=True)
            for b in range(k):
                pltpu.make_async_copy(rows_v.at[b],
                                      acc_sh.at[dst_v.at[g + b]],
                                      ssem).wait()
            return carry

        lax.fori_loop(0, nch // k, lambda g, c2: body(g * k, c2), 0,
                      unroll=False)
        plsc.subcore_barrier()

        @pl.when(s == 0)
        def _():
            pltpu.sync_copy(acc_sh, out_hbm.at[c])

    return agg_kernel


def _xw_body(h_ref, w_ref, deg_ref, z_ref, dis_ref):
    deg = deg_ref[0] + deg_ref[1] + 1.0
    dis = lax.rsqrt(deg)
    xw = jnp.dot(h_ref[...], w_ref[...], preferred_element_type=jnp.float32)
    z_ref[...] = dis * xw
    dis_ref[...] = dis


def _loss_body(agg_ref, z_ref, dis_ref, y_ref, b_ref, out_ref):
    n, ncls = z_ref.shape
    dis = dis_ref[...]
    out = dis * (agg_ref[0] + agg_ref[1] + z_ref[...]) + b_ref[...]
    m = jnp.max(out, axis=1, keepdims=True)
    lse = m + jnp.log(jnp.sum(jnp.exp(out - m), axis=1, keepdims=True))
    cls_ids = lax.broadcasted_iota(jnp.int32, (n, ncls), 1)
    picked = jnp.sum(jnp.where(cls_ids == y_ref[...], out, 0.0),
                     axis=1, keepdims=True)
    out_ref[...] = jnp.sum(lse - picked, axis=0, keepdims=True) / n


def kernel(hiddens, edge_index, y, W, b):
    n, hid = hiddens.shape
    ncls = W.shape[1]
    e = edge_index.shape[1]
    ch = 125
    nch = e // (NW * ch)
    assert NW * nch * ch == e

    src = edge_index[0].reshape(NW, nch, ch)
    dst = edge_index[1].reshape(NW, nch, ch)
    ones_ch = jnp.ones((ch,), jnp.float32)
    zeros_n = jnp.zeros((n,), jnp.float32)
    zeros_n8 = jnp.zeros((n, ncls), jnp.float32)

    deg2 = _make_deg_kernel(n, nch, ch)(dst, ones_ch, zeros_n)

    z, dis = pl.pallas_call(
        _xw_body,
        out_shape=[
            jax.ShapeDtypeStruct((n, ncls), jnp.float32),
            jax.ShapeDtypeStruct((n, 1), jnp.float32),
        ],
    )(hiddens, W, deg2.reshape(NC, n, 1))

    agg2 = _make_agg_kernel(n, ncls, nch, ch)(src, dst, z, zeros_n8)

    loss = pl.pallas_call(
        _loss_body,
        out_shape=jax.ShapeDtypeStruct((1, 1), jnp.float32),
    )(agg2, z, dis, y.reshape(n, 1), b.reshape(1, ncls))

    return loss.reshape(())


# speedup vs baseline: 100.4283x; 1.5202x over previous
import functools

import jax
import jax.numpy as jnp
from jax import lax
from jax.experimental import pallas as pl
from jax.experimental.pallas import tpu as pltpu
from jax.experimental.pallas import tpu_sc as plsc

NC = 2
NS = 16
NW = NC * NS
CH = 128
K = 6


def _sc_mesh():
    return plsc.VectorSubcoreMesh(core_axis_name="c", subcore_axis_name="s")


def _make_deg_kernel(npad, nchw, ntail):
    @functools.partial(
        pl.kernel,
        out_type=jax.ShapeDtypeStruct((NC, npad), jnp.float32),
        mesh=_sc_mesh(),
        compiler_params=pltpu.CompilerParams(use_tc_tiling_on_sc=False),
        scratch_types=[
            pltpu.VMEM((nchw, CH), jnp.int32),
            pltpu.VMEM((1, CH), jnp.int32),
            pltpu.VMEM((CH,), jnp.float32),
            pltpu.VMEM_SHARED((npad,), jnp.float32),
            pltpu.SemaphoreType.DMA,
        ],
    )
    def deg_kernel(dst_hbm, dtail_hbm, ones_hbm, zeros_hbm, out_hbm,
                   idx_v, tidx_v, ones_v, acc_sh, sem):
        c = lax.axis_index("c")
        s = lax.axis_index("s")
        wid = c * NS + s
        pltpu.sync_copy(dst_hbm.at[pl.ds(wid * nchw, nchw)], idx_v)
        pltpu.sync_copy(ones_hbm, ones_v)

        @pl.when(s == 0)
        def _():
            pltpu.sync_copy(zeros_hbm, acc_sh)

        @pl.when(wid < ntail)
        def _():
            pltpu.sync_copy(dtail_hbm.at[pl.ds(wid, 1)], tidx_v)

        plsc.subcore_barrier()

        def body(g, carry):
            for b in range(K):
                pltpu.async_copy(ones_v, acc_sh.at[idx_v.at[g + b]], sem,
                                 add=True)
            for b in range(K):
                pltpu.make_async_copy(ones_v, acc_sh.at[idx_v.at[g + b]],
                                      sem).wait()
            return carry

        lax.fori_loop(0, nchw // K, lambda g, cr: body(g * K, cr), 0,
                      unroll=False)

        @pl.when(wid < ntail)
        def _():
            pltpu.sync_copy(ones_v, acc_sh.at[tidx_v.at[0]], add=True)

        plsc.subcore_barrier()

        @pl.when(s == 0)
        def _():
            pltpu.sync_copy(acc_sh, out_hbm.at[c])

    return deg_kernel


def _make_agg_kernel(npad, ncls, nchw, ntail):
    @functools.partial(
        pl.kernel,
        out_type=jax.ShapeDtypeStruct((NC, npad, ncls), jnp.float32),
        mesh=_sc_mesh(),
        compiler_params=pltpu.CompilerParams(use_tc_tiling_on_sc=False),
        scratch_types=[
            pltpu.VMEM((nchw, CH), jnp.int32),
            pltpu.VMEM((nchw, CH), jnp.int32),
            pltpu.VMEM((1, CH), jnp.int32),
            pltpu.VMEM((1, CH), jnp.int32),
            pltpu.VMEM((K, CH, ncls), jnp.float32),
            pltpu.VMEM((CH, ncls), jnp.float32),
            pltpu.VMEM_SHARED((npad, ncls), jnp.float32),
            pltpu.SemaphoreType.DMA,
            pltpu.SemaphoreType.DMA,
        ],
    )
    def agg_kernel(src_hbm, dst_hbm, stail_hbm, dtail_hbm, z_hbm, zeros_hbm,
                   out_hbm, src_v, dst_v, ts_v, td_v, rows_v, trows_v,
                   acc_sh, gsem, ssem):
        c = lax.axis_index("c")
        s = lax.axis_index("s")
        wid = c * NS + s
        pltpu.sync_copy(src_hbm.at[pl.ds(wid * nchw, nchw)], src_v)
        pltpu.sync_copy(dst_hbm.at[pl.ds(wid * nchw, nchw)], dst_v)

        @pl.when(s == 0)
        def _():
            pltpu.sync_copy(zeros_hbm, acc_sh)

        @pl.when(wid < ntail)
        def _():
            pltpu.sync_copy(stail_hbm.at[pl.ds(wid, 1)], ts_v)
            pltpu.sync_copy(dtail_hbm.at[pl.ds(wid, 1)], td_v)

        plsc.subcore_barrier()

        def body(g, carry):
            for b in range(K):
                pltpu.async_copy(z_hbm.at[src_v.at[g + b]], rows_v.at[b],
                                 gsem)
            for b in range(K):
                pltpu.make_async_copy(z_hbm.at[src_v.at[g + b]],
                                      rows_v.at[b], gsem).wait()
            for b in range(K):
                pltpu.async_copy(rows_v.at[b], acc_sh.at[dst_v.at[g + b]],
                                 ssem, add=True)
            for b in range(K):
                pltpu.make_async_copy(rows_v.at[b],
                                      acc_sh.at[dst_v.at[g + b]],
                                      ssem).wait()
            return carry

        lax.fori_loop(0, nchw // K, lambda g, cr: body(g * K, cr), 0,
                      unroll=False)

        @pl.when(wid < ntail)
        def _():
            pltpu.async_copy(z_hbm.at[ts_v.at[0]], trows_v, gsem).wait()
            pltpu.sync_copy(trows_v, acc_sh.at[td_v.at[0]], add=True)

        plsc.subcore_barrier()

        @pl.when(s == 0)
        def _():
            pltpu.sync_copy(acc_sh, out_hbm.at[c])

    return agg_kernel


def _xw_body(h2_ref, w2_ref, deg_ref, r_ref, z_ref, de_ref):
    deg16 = deg_ref[0] + deg_ref[1] + 1.0
    dis16 = lax.rsqrt(deg16)
    dis_exp = jnp.dot(dis16, r_ref[...],
                      preferred_element_type=jnp.float32)
    xw = jnp.dot(h2_ref[...], w2_ref[...],
                 preferred_element_type=jnp.float32)
    z_ref[...] = dis_exp * xw
    de_ref[...] = dis_exp


def _loss_body(agg_ref, z_ref, de_ref, y_ref, b_ref, r_ref, out_ref):
    nr, nl = z_ref.shape
    out = de_ref[...] * (agg_ref[0] + agg_ref[1] + z_ref[...]) + b_ref[...]
    valid = lax.broadcasted_iota(jnp.int32, (nr, nl), 0) < 625
    validf = valid.astype(jnp.float32)
    gmax = jnp.max(jnp.where(valid, out, -jnp.inf))
    ex = jnp.exp(out - gmax) * validf
    sums16 = lax.dot_general(ex, r_ref[...], (((1,), (1,)), ((), ())),
                             preferred_element_type=jnp.float32)
    valid16 = lax.broadcasted_iota(jnp.int32, sums16.shape, 0) < 625
    lse_sum = jnp.sum(jnp.log(sums16 + (1.0 - valid16.astype(jnp.float32))))
    y_exp = jnp.dot(y_ref[...].astype(jnp.float32), r_ref[...],
                    preferred_element_type=jnp.float32)
    cls_f = (lax.broadcasted_iota(jnp.int32, (nr, nl), 1) % 8
             ).astype(jnp.float32)
    picked = jnp.sum(jnp.where(cls_f == y_exp, out, 0.0) * validf)
    n = 10000.0
    loss = (n * gmax + lse_sum - picked) / n
    out_ref[...] = jnp.full((1, 1), loss, jnp.float32)


def kernel(hiddens, edge_index, y, W, b):
    n, hid = hiddens.shape
    ncls = W.shape[1]
    e = edge_index.shape[1]
    npad = 10240
    nchw = 78
    emain = NW * nchw * CH
    ntail = (e - emain) // CH
    assert emain + ntail * CH == e and ncls == 8 and n == 10000

    src_m = edge_index[0, :emain].reshape(NW * nchw, CH)
    dst_m = edge_index[1, :emain].reshape(NW * nchw, CH)
    src_t = edge_index[0, emain:].reshape(ntail, CH)
    dst_t = edge_index[1, emain:].reshape(ntail, CH)
    ones_ch = jnp.ones((CH,), jnp.float32)
    zeros_n = jnp.zeros((npad,), jnp.float32)
    zeros_n8 = jnp.zeros((npad, ncls), jnp.float32)

    R = (jnp.eye(16, dtype=jnp.float32)[:, :, None]
         * jnp.ones((ncls,), jnp.float32)).reshape(16, 128)
    W2 = (jnp.eye(16, dtype=jnp.float32)[:, None, :, None]
          * W[None, :, None, :]).reshape(16 * hid, 128)
    h2 = jnp.pad(hiddens.reshape(n // 16, 16 * hid), ((0, 15), (0, 0)))
    y16 = jnp.pad(y, (0, npad - n)).reshape(npad // 16, 16)
    b128 = jnp.tile(b, 16).reshape(1, 128)

    deg2 = _make_deg_kernel(npad, nchw, ntail)(dst_m, dst_t, ones_ch,
                                               zeros_n)

    z, dis_exp = pl.pallas_call(
        _xw_body,
        out_shape=[
            jax.ShapeDtypeStruct((npad // 16, 128), jnp.float32),
            jax.ShapeDtypeStruct((npad // 16, 128), jnp.float32),
        ],
    )(h2, W2, deg2.reshape(NC, npad // 16, 16), R)

    agg2 = _make_agg_kernel(npad, ncls, nchw, ntail)(
        src_m, dst_m, src_t, dst_t, z.reshape(npad, ncls), zeros_n8)

    loss = pl.pallas_call(
        _loss_body,
        out_shape=jax.ShapeDtypeStruct((1, 1), jnp.float32),
    )(agg2.reshape(NC, npad // 16, 128), z, dis_exp, y16, b128, R)

    return loss.reshape(())


# speedup vs baseline: 127.4121x; 1.2687x over previous
import functools

import jax
import jax.numpy as jnp
from jax import lax
from jax.experimental import pallas as pl
from jax.experimental.pallas import tpu as pltpu
from jax.experimental.pallas import tpu_sc as plsc

NC = 2
NS = 16
NW = NC * NS
CH = 128
K = 6


def _sc_mesh():
    return plsc.VectorSubcoreMesh(core_axis_name="c", subcore_axis_name="s")


def _make_deg_kernel(npad, nchw, ntail):
    @functools.partial(
        pl.kernel,
        out_type=jax.ShapeDtypeStruct((NC, npad), jnp.float32),
        mesh=_sc_mesh(),
        compiler_params=pltpu.CompilerParams(use_tc_tiling_on_sc=False),
        scratch_types=[
            pltpu.VMEM((nchw, CH), jnp.int32),
            pltpu.VMEM((1, CH), jnp.int32),
            pltpu.VMEM((CH,), jnp.float32),
            pltpu.VMEM_SHARED((npad,), jnp.float32),
            pltpu.SemaphoreType.DMA,
        ],
    )
    def deg_kernel(edge_hbm, ones_hbm, zeros_hbm, out_hbm,
                   idx_v, tidx_v, ones_v, acc_sh, sem):
        c = lax.axis_index("c")
        s = lax.axis_index("s")
        wid = c * NS + s
        pltpu.sync_copy(edge_hbm.at[1].at[pl.ds(wid * nchw, nchw)], idx_v)
        pltpu.sync_copy(ones_hbm, ones_v)

        @pl.when(s == 0)
        def _():
            pltpu.sync_copy(zeros_hbm, acc_sh)

        @pl.when(wid < ntail)
        def _():
            pltpu.sync_copy(edge_hbm.at[1].at[pl.ds(NW * nchw + wid, 1)],
                            tidx_v)

        plsc.subcore_barrier()

        def body(g, carry):
            for b in range(K):
                pltpu.async_copy(ones_v, acc_sh.at[idx_v.at[g + b]], sem,
                                 add=True)
            for b in range(K):
                pltpu.make_async_copy(ones_v, acc_sh.at[idx_v.at[g + b]],
                                      sem).wait()
            return carry

        lax.fori_loop(0, nchw // K, lambda g, cr: body(g * K, cr), 0,
                      unroll=False)

        @pl.when(wid < ntail)
        def _():
            pltpu.sync_copy(ones_v, acc_sh.at[tidx_v.at[0]], add=True)

        plsc.subcore_barrier()

        @pl.when(s == 0)
        def _():
            pltpu.sync_copy(acc_sh, out_hbm.at[c])

    return deg_kernel


def _make_agg_kernel(npad, ncls, nchw, ntail):
    @functools.partial(
        pl.kernel,
        out_type=jax.ShapeDtypeStruct((NC, npad, ncls), jnp.float32),
        mesh=_sc_mesh(),
        compiler_params=pltpu.CompilerParams(use_tc_tiling_on_sc=False),
        scratch_types=[
            pltpu.VMEM((nchw, CH), jnp.int32),
            pltpu.VMEM((nchw, CH), jnp.int32),
            pltpu.VMEM((1, CH), jnp.int32),
            pltpu.VMEM((1, CH), jnp.int32),
            pltpu.VMEM((K, CH, ncls), jnp.float32),
            pltpu.VMEM((CH, ncls), jnp.float32),
            pltpu.VMEM_SHARED((npad, ncls), jnp.float32),
            pltpu.VMEM_SHARED((npad, ncls), jnp.float32),
            pltpu.SemaphoreType.DMA,
            pltpu.SemaphoreType.DMA,
        ],
    )
    def agg_kernel(edge_hbm, z_hbm, zeros_hbm,
                   out_hbm, src_v, dst_v, ts_v, td_v, rows_v, trows_v,
                   acc_sh, z_sh, gsem, ssem):
        c = lax.axis_index("c")
        s = lax.axis_index("s")
        wid = c * NS + s
        pltpu.sync_copy(edge_hbm.at[0].at[pl.ds(wid * nchw, nchw)], src_v)
        pltpu.sync_copy(edge_hbm.at[1].at[pl.ds(wid * nchw, nchw)], dst_v)

        @pl.when(s == 0)
        def _():
            pltpu.sync_copy(zeros_hbm, acc_sh)

        @pl.when(s == 1)
        def _():
            pltpu.sync_copy(z_hbm, z_sh)

        @pl.when(wid < ntail)
        def _():
            pltpu.sync_copy(edge_hbm.at[0].at[pl.ds(NW * nchw + wid, 1)],
                            ts_v)
            pltpu.sync_copy(edge_hbm.at[1].at[pl.ds(NW * nchw + wid, 1)],
                            td_v)

        plsc.subcore_barrier()

        def body(g, carry):
            for b in range(K):
                pltpu.async_copy(z_sh.at[src_v.at[g + b]], rows_v.at[b],
                                 gsem)
            for b in range(K):
                pltpu.make_async_copy(z_sh.at[src_v.at[g + b]],
                                      rows_v.at[b], gsem).wait()
            for b in range(K):
                pltpu.async_copy(rows_v.at[b], acc_sh.at[dst_v.at[g + b]],
                                 ssem, add=True)
            for b in range(K):
                pltpu.make_async_copy(rows_v.at[b],
                                      acc_sh.at[dst_v.at[g + b]],
                                      ssem).wait()
            return carry

        lax.fori_loop(0, nchw // K, lambda g, cr: body(g * K, cr), 0,
                      unroll=False)

        @pl.when(wid < ntail)
        def _():
            pltpu.async_copy(z_sh.at[ts_v.at[0]], trows_v, gsem).wait()
            pltpu.sync_copy(trows_v, acc_sh.at[td_v.at[0]], add=True)

        plsc.subcore_barrier()

        @pl.when(s == 0)
        def _():
            pltpu.sync_copy(acc_sh, out_hbm.at[c])

    return agg_kernel


def _xw_body(h2_ref, w2_ref, xw_ref):
    xw_ref[...] = jnp.dot(h2_ref[...], w2_ref[...],
                          preferred_element_type=jnp.float32)


def _z_body(xw_ref, deg_ref, r_ref, z_ref, de_ref):
    deg16 = deg_ref[0] + deg_ref[1] + 1.0
    dis16 = lax.rsqrt(deg16)
    dis_exp = jnp.dot(dis16, r_ref[...],
                      preferred_element_type=jnp.float32)
    z_ref[...] = dis_exp * xw_ref[...]
    de_ref[...] = dis_exp


def _loss_body(agg_ref, z_ref, de_ref, y_ref, b_ref, r_ref, out_ref):
    nr, nl = z_ref.shape
    out = de_ref[...] * (agg_ref[0] + agg_ref[1] + z_ref[...]) + b_ref[...]
    valid = lax.broadcasted_iota(jnp.int32, (nr, nl), 0) < 625
    validf = valid.astype(jnp.float32)
    gmax = jnp.max(jnp.where(valid, out, -jnp.inf))
    ex = jnp.exp(out - gmax) * validf
    sums16 = lax.dot_general(ex, r_ref[...], (((1,), (1,)), ((), ())),
                             preferred_element_type=jnp.float32)
    valid16 = lax.broadcasted_iota(jnp.int32, sums16.shape, 0) < 625
    lse_sum = jnp.sum(jnp.log(sums16 + (1.0 - valid16.astype(jnp.float32))))
    y_exp = jnp.dot(y_ref[...].astype(jnp.float32), r_ref[...],
                    preferred_element_type=jnp.float32)
    cls_f = (lax.broadcasted_iota(jnp.int32, (nr, nl), 1) % 8
             ).astype(jnp.float32)
    picked = jnp.sum(jnp.where(cls_f == y_exp, out, 0.0) * validf)
    n = 10000.0
    loss = (n * gmax + lse_sum - picked) / n
    out_ref[...] = jnp.full((1, 1), loss, jnp.float32)


def kernel(hiddens, edge_index, y, W, b):
    n, hid = hiddens.shape
    ncls = W.shape[1]
    e = edge_index.shape[1]
    npad = 10240
    nchw = 78
    emain = NW * nchw * CH
    ntail = (e - emain) // CH
    assert emain + ntail * CH == e and ncls == 8 and n == 10000

    edge3 = edge_index.reshape(2, e // CH, CH)
    ones_ch = jnp.ones((CH,), jnp.float32)
    zeros_n = jnp.zeros((npad,), jnp.float32)
    zeros_n8 = jnp.zeros((npad, ncls), jnp.float32)

    R = (jnp.eye(16, dtype=jnp.float32)[:, :, None]
         * jnp.ones((ncls,), jnp.float32)).reshape(16, 128)
    W2 = (jnp.eye(16, dtype=jnp.float32)[:, None, :, None]
          * W[None, :, None, :]).reshape(16 * hid, 128)
    h2 = jnp.pad(hiddens.reshape(n // 16, 16 * hid), ((0, 15), (0, 0)))
    y16 = jnp.pad(y, (0, npad - n)).reshape(npad // 16, 16)
    b128 = jnp.tile(b, 16).reshape(1, 128)

    deg2 = _make_deg_kernel(npad, nchw, ntail)(edge3, ones_ch, zeros_n)

    xw = pl.pallas_call(
        _xw_body,
        out_shape=jax.ShapeDtypeStruct((npad // 16, 128), jnp.float32),
    )(h2, W2)

    z, dis_exp = pl.pallas_call(
        _z_body,
        out_shape=[
            jax.ShapeDtypeStruct((npad // 16, 128), jnp.float32),
            jax.ShapeDtypeStruct((npad // 16, 128), jnp.float32),
        ],
    )(xw, deg2.reshape(NC, npad // 16, 16), R)

    agg2 = _make_agg_kernel(npad, ncls, nchw, ntail)(
        edge3, z.reshape(npad, ncls), zeros_n8)

    loss = pl.pallas_call(
        _loss_body,
        out_shape=jax.ShapeDtypeStruct((1, 1), jnp.float32),
    )(agg2.reshape(NC, npad // 16, 128), z, dis_exp, y16, b128, R)

    return loss.reshape(())


# speedup vs baseline: 133.5817x; 1.0484x over previous
import functools

import jax
import jax.numpy as jnp
from jax import lax
from jax.experimental import pallas as pl
from jax.experimental.pallas import tpu as pltpu
from jax.experimental.pallas import tpu_sc as plsc

NC = 2
NS = 16
NW = NC * NS
CH = 128
K = 6


def _sc_mesh():
    return plsc.VectorSubcoreMesh(core_axis_name="c", subcore_axis_name="s")


def _make_deg_kernel(npad, ncls, nchw, ntail):
    @functools.partial(
        pl.kernel,
        out_type=jax.ShapeDtypeStruct((NC, npad, ncls), jnp.float32),
        mesh=_sc_mesh(),
        compiler_params=pltpu.CompilerParams(use_tc_tiling_on_sc=False),
        scratch_types=[
            pltpu.VMEM((nchw, CH), jnp.int32),
            pltpu.VMEM((1, CH), jnp.int32),
            pltpu.VMEM((CH, ncls), jnp.float32),
            pltpu.VMEM_SHARED((npad, ncls), jnp.float32),
            pltpu.SemaphoreType.DMA,
        ],
    )
    def deg_kernel(edge_hbm, ones_hbm, zeros_hbm, out_hbm,
                   idx_v, tidx_v, ones_v, acc_sh, sem):
        c = lax.axis_index("c")
        s = lax.axis_index("s")
        wid = c * NS + s
        pltpu.sync_copy(edge_hbm.at[1].at[pl.ds(wid * nchw, nchw)], idx_v)
        pltpu.sync_copy(ones_hbm, ones_v)

        @pl.when(s == 0)
        def _():
            pltpu.sync_copy(zeros_hbm, acc_sh)

        @pl.when(wid < ntail)
        def _():
            pltpu.sync_copy(edge_hbm.at[1].at[pl.ds(NW * nchw + wid, 1)],
                            tidx_v)

        plsc.subcore_barrier()

        def body(g, carry):
            for b in range(K):
                pltpu.async_copy(ones_v, acc_sh.at[idx_v.at[g + b]], sem,
                                 add=True)
            for b in range(K):
                pltpu.make_async_copy(ones_v, acc_sh.at[idx_v.at[g + b]],
                                      sem).wait()
            return carry

        lax.fori_loop(0, nchw // K, lambda g, cr: body(g * K, cr), 0,
                      unroll=False)

        @pl.when(wid < ntail)
        def _():
            pltpu.sync_copy(ones_v, acc_sh.at[tidx_v.at[0]], add=True)

        plsc.subcore_barrier()

        @pl.when(s == 0)
        def _():
            pltpu.sync_copy(acc_sh, out_hbm.at[c])

    return deg_kernel


def _make_agg_kernel(npad, ncls, nchw, ntail):
    @functools.partial(
        pl.kernel,
        out_type=jax.ShapeDtypeStruct((NC, npad, ncls), jnp.float32),
        mesh=_sc_mesh(),
        compiler_params=pltpu.CompilerParams(use_tc_tiling_on_sc=False),
        scratch_types=[
            pltpu.VMEM((nchw, CH), jnp.int32),
            pltpu.VMEM((nchw, CH), jnp.int32),
            pltpu.VMEM((1, CH), jnp.int32),
            pltpu.VMEM((1, CH), jnp.int32),
            pltpu.VMEM((K, CH, ncls), jnp.float32),
            pltpu.VMEM((CH, ncls), jnp.float32),
            pltpu.VMEM_SHARED((npad, ncls), jnp.float32),
            pltpu.VMEM_SHARED((npad, ncls), jnp.float32),
            pltpu.SemaphoreType.DMA,
            pltpu.SemaphoreType.DMA,
        ],
    )
    def agg_kernel(edge_hbm, z_hbm, zeros_hbm,
                   out_hbm, src_v, dst_v, ts_v, td_v, rows_v, trows_v,
                   acc_sh, z_sh, gsem, ssem):
        c = lax.axis_index("c")
        s = lax.axis_index("s")
        wid = c * NS + s
        pltpu.sync_copy(edge_hbm.at[0].at[pl.ds(wid * nchw, nchw)], src_v)
        pltpu.sync_copy(edge_hbm.at[1].at[pl.ds(wid * nchw, nchw)], dst_v)

        @pl.when(s == 0)
        def _():
            pltpu.sync_copy(zeros_hbm, acc_sh)

        @pl.when(s == 1)
        def _():
            pltpu.sync_copy(z_hbm, z_sh)

        @pl.when(wid < ntail)
        def _():
            pltpu.sync_copy(edge_hbm.at[0].at[pl.ds(NW * nchw + wid, 1)],
                            ts_v)
            pltpu.sync_copy(edge_hbm.at[1].at[pl.ds(NW * nchw + wid, 1)],
                            td_v)

        plsc.subcore_barrier()

        def body(g, carry):
            for b in range(K):
                pltpu.async_copy(z_sh.at[src_v.at[g + b]], rows_v.at[b],
                                 gsem)
            for b in range(K):
                pltpu.make_async_copy(z_sh.at[src_v.at[g + b]],
                                      rows_v.at[b], gsem).wait()
            for b in range(K):
                pltpu.async_copy(rows_v.at[b], acc_sh.at[dst_v.at[g + b]],
                                 ssem, add=True)
            for b in range(K):
                pltpu.make_async_copy(rows_v.at[b],
                                      acc_sh.at[dst_v.at[g + b]],
                                      ssem).wait()
            return carry

        lax.fori_loop(0, nchw // K, lambda g, cr: body(g * K, cr), 0,
                      unroll=False)

        @pl.when(wid < ntail)
        def _():
            pltpu.async_copy(z_sh.at[ts_v.at[0]], trows_v, gsem).wait()
            pltpu.sync_copy(trows_v, acc_sh.at[td_v.at[0]], add=True)

        plsc.subcore_barrier()

        @pl.when(s == 0)
        def _():
            pltpu.sync_copy(acc_sh, out_hbm.at[c])

    return agg_kernel


def _xw_body(h2_ref, w2_ref, xw_ref):
    xw_ref[...] = jnp.dot(h2_ref[...], w2_ref[...],
                          preferred_element_type=jnp.float32)


def _z_body(xw_ref, deg_ref, z_ref, de_ref):
    deg_exp = deg_ref[0] + deg_ref[1] + 1.0
    dis_exp = lax.rsqrt(deg_exp)
    z_ref[...] = dis_exp * xw_ref[...]
    de_ref[...] = dis_exp


def _loss_body(agg_ref, z_ref, de_ref, y_ref, b_ref, r_ref, out_ref):
    nr, nl = z_ref.shape
    out = de_ref[...] * (agg_ref[0] + agg_ref[1] + z_ref[...]) + b_ref[...]
    valid = lax.broadcasted_iota(jnp.int32, (nr, nl), 0) < 625
    validf = valid.astype(jnp.float32)
    gmax = jnp.max(jnp.where(valid, out, -jnp.inf))
    ex = jnp.exp(out - gmax) * validf
    sums16 = lax.dot_general(ex, r_ref[...], (((1,), (1,)), ((), ())),
                             preferred_element_type=jnp.float32)
    valid16 = lax.broadcasted_iota(jnp.int32, sums16.shape, 0) < 625
    lse_sum = jnp.sum(jnp.log(sums16 + (1.0 - valid16.astype(jnp.float32))))
    y_exp = jnp.dot(y_ref[...].astype(jnp.float32), r_ref[...],
                    preferred_element_type=jnp.float32)
    cls_f = (lax.broadcasted_iota(jnp.int32, (nr, nl), 1) % 8
             ).astype(jnp.float32)
    picked = jnp.sum(jnp.where(cls_f == y_exp, out, 0.0) * validf)
    n = 10000.0
    loss = (n * gmax + lse_sum - picked) / n
    out_ref[...] = jnp.full((1, 1), loss, jnp.float32)


def kernel(hiddens, edge_index, y, W, b):
    n, hid = hiddens.shape
    ncls = W.shape[1]
    e = edge_index.shape[1]
    npad = 10240
    nchw = 78
    emain = NW * nchw * CH
    ntail = (e - emain) // CH
    assert emain + ntail * CH == e and ncls == 8 and n == 10000

    edge3 = edge_index.reshape(2, e // CH, CH)
    ones_ch8 = jnp.ones((CH, ncls), jnp.float32)
    zeros_n8 = jnp.zeros((npad, ncls), jnp.float32)

    R = (jnp.eye(16, dtype=jnp.float32)[:, :, None]
         * jnp.ones((ncls,), jnp.float32)).reshape(16, 128)
    y16 = jnp.pad(y, (0, npad - n)).reshape(npad // 16, 16)
    b128 = jnp.tile(b, 16).reshape(1, 128)
    W2 = (jnp.eye(16, dtype=jnp.float32)[:, None, :, None]
          * W[None, :, None, :]).reshape(16 * hid, 128)
    h2 = jnp.pad(hiddens.reshape(n // 16, 16 * hid), ((0, 15), (0, 0)))

    deg2 = _make_deg_kernel(npad, ncls, nchw, ntail)(edge3, ones_ch8,
                                                     zeros_n8)

    xw = pl.pallas_call(
        _xw_body,
        out_shape=jax.ShapeDtypeStruct((npad // 16, 128), jnp.float32),
    )(h2, W2)

    z, dis_exp = pl.pallas_call(
        _z_body,
        out_shape=[
            jax.ShapeDtypeStruct((npad // 16, 128), jnp.float32),
            jax.ShapeDtypeStruct((npad // 16, 128), jnp.float32),
        ],
    )(xw, deg2.reshape(NC, npad // 16, 128))

    agg2 = _make_agg_kernel(npad, ncls, nchw, ntail)(
        edge3, z.reshape(npad, ncls), zeros_n8)

    loss = pl.pallas_call(
        _loss_body,
        out_shape=jax.ShapeDtypeStruct((1, 1), jnp.float32),
    )(agg2.reshape(NC, npad // 16, 128), z, dis_exp, y16, b128, R)

    return loss.reshape(())


# speedup vs baseline: 137.6726x; 1.0306x over previous
import functools

import jax
import jax.numpy as jnp
from jax import lax
from jax.experimental import pallas as pl
from jax.experimental.pallas import tpu as pltpu
from jax.experimental.pallas import tpu_sc as plsc

NC = 2
NS = 16
NW = NC * NS
CH = 128
K = 6


def _sc_mesh():
    return plsc.VectorSubcoreMesh(core_axis_name="c", subcore_axis_name="s")


def _make_deg_kernel(npad, ncls, nchw, ntail):
    @functools.partial(
        pl.kernel,
        out_type=jax.ShapeDtypeStruct((NC, npad, ncls), jnp.float32),
        mesh=_sc_mesh(),
        compiler_params=pltpu.CompilerParams(use_tc_tiling_on_sc=False),
        scratch_types=[
            pltpu.VMEM((nchw, CH), jnp.int32),
            pltpu.VMEM((1, CH), jnp.int32),
            pltpu.VMEM((CH, ncls), jnp.float32),
            pltpu.VMEM_SHARED((npad, ncls), jnp.float32),
            pltpu.SemaphoreType.DMA,
        ],
    )
    def deg_kernel(edge_hbm, ones_hbm, zeros_hbm, out_hbm,
                   idx_v, tidx_v, ones_v, acc_sh, sem):
        c = lax.axis_index("c")
        s = lax.axis_index("s")
        wid = c * NS + s
        pltpu.sync_copy(edge_hbm.at[1].at[pl.ds(wid * nchw, nchw)], idx_v)
        pltpu.sync_copy(ones_hbm, ones_v)

        @pl.when(s == 0)
        def _():
            pltpu.sync_copy(zeros_hbm, acc_sh)

        @pl.when(wid < ntail)
        def _():
            pltpu.sync_copy(edge_hbm.at[1].at[pl.ds(NW * nchw + wid, 1)],
                            tidx_v)

        plsc.subcore_barrier()

        def body(g, carry):
            for b in range(K):
                pltpu.async_copy(ones_v, acc_sh.at[idx_v.at[g + b]], sem,
                                 add=True)
            for b in range(K):
                pltpu.make_async_copy(ones_v, acc_sh.at[idx_v.at[g + b]],
                                      sem).wait()
            return carry

        lax.fori_loop(0, nchw // K, lambda g, cr: body(g * K, cr), 0,
                      unroll=False)

        @pl.when(wid < ntail)
        def _():
            pltpu.sync_copy(ones_v, acc_sh.at[tidx_v.at[0]], add=True)

        plsc.subcore_barrier()

        @pl.when(s == 0)
        def _():
            pltpu.sync_copy(acc_sh, out_hbm.at[c])

    return deg_kernel


def _make_agg_kernel(npad, ncls, nchw, ntail):
    @functools.partial(
        pl.kernel,
        out_type=jax.ShapeDtypeStruct((NC, npad, ncls), jnp.float32),
        mesh=_sc_mesh(),
        compiler_params=pltpu.CompilerParams(use_tc_tiling_on_sc=False),
        scratch_types=[
            pltpu.VMEM((nchw, CH), jnp.int32),
            pltpu.VMEM((nchw, CH), jnp.int32),
            pltpu.VMEM((1, CH), jnp.int32),
            pltpu.VMEM((1, CH), jnp.int32),
            pltpu.VMEM((2, K, CH, ncls), jnp.float32),
            pltpu.VMEM((CH, ncls), jnp.float32),
            pltpu.VMEM_SHARED((npad, ncls), jnp.float32),
            pltpu.VMEM_SHARED((npad, ncls), jnp.float32),
            pltpu.SemaphoreType.DMA,
            pltpu.SemaphoreType.DMA,
        ],
    )
    def agg_kernel(edge_hbm, z_hbm, zeros_hbm,
                   out_hbm, src_v, dst_v, ts_v, td_v, rows_v, trows_v,
                   acc_sh, z_sh, gsem, ssem):
        c = lax.axis_index("c")
        s = lax.axis_index("s")
        wid = c * NS + s
        pltpu.sync_copy(edge_hbm.at[0].at[pl.ds(wid * nchw, nchw)], src_v)
        pltpu.sync_copy(edge_hbm.at[1].at[pl.ds(wid * nchw, nchw)], dst_v)

        @pl.when(s == 0)
        def _():
            pltpu.sync_copy(zeros_hbm, acc_sh)

        @pl.when(s == 1)
        def _():
            pltpu.sync_copy(z_hbm, z_sh)

        @pl.when(wid < ntail)
        def _():
            pltpu.sync_copy(edge_hbm.at[0].at[pl.ds(NW * nchw + wid, 1)],
                            ts_v)
            pltpu.sync_copy(edge_hbm.at[1].at[pl.ds(NW * nchw + wid, 1)],
                            td_v)

        plsc.subcore_barrier()

        ng = nchw // K

        def fire_gathers(g, p):
            for b in range(K):
                pltpu.async_copy(z_sh.at[src_v.at[g * K + b]],
                                 rows_v.at[p].at[b], gsem)

        def drain_gathers(g, p):
            for b in range(K):
                pltpu.make_async_copy(z_sh.at[src_v.at[g * K + b]],
                                      rows_v.at[p].at[b], gsem).wait()

        def fire_scatters(g, p):
            for b in range(K):
                pltpu.async_copy(rows_v.at[p].at[b],
                                 acc_sh.at[dst_v.at[g * K + b]],
                                 ssem, add=True)

        def drain_scatters(g, p):
            for b in range(K):
                pltpu.make_async_copy(rows_v.at[p].at[b],
                                      acc_sh.at[dst_v.at[g * K + b]],
                                      ssem).wait()

        fire_gathers(0, 0)

        def body(g, carry):
            p = lax.rem(g, 2)
            drain_gathers(g, p)

            @pl.when(g < ng - 1)
            def _():
                @pl.when(g >= 1)
                def _():
                    drain_scatters(g - 1, 1 - p)

                fire_gathers(g + 1, 1 - p)

            fire_scatters(g, p)
            return carry

        lax.fori_loop(0, ng, body, 0, unroll=False)
        drain_scatters(ng - 2, (ng - 2) % 2)
        drain_scatters(ng - 1, (ng - 1) % 2)

        @pl.when(wid < ntail)
        def _():
            pltpu.async_copy(z_sh.at[ts_v.at[0]], trows_v, gsem).wait()
            pltpu.sync_copy(trows_v, acc_sh.at[td_v.at[0]], add=True)

        plsc.subcore_barrier()

        @pl.when(s == 0)
        def _():
            pltpu.sync_copy(acc_sh, out_hbm.at[c])

    return agg_kernel


def _xw_body(h2_ref, w2_ref, xw_ref):
    xw_ref[...] = jnp.dot(h2_ref[...], w2_ref[...],
                          preferred_element_type=jnp.float32)


def _z_body(xw_ref, deg_ref, z_ref, de_ref):
    deg_exp = deg_ref[0] + deg_ref[1] + 1.0
    dis_exp = lax.rsqrt(deg_exp)
    z_ref[...] = dis_exp * xw_ref[...]
    de_ref[...] = dis_exp


def _loss_body(agg_ref, z_ref, de_ref, y_ref, b_ref, r_ref, out_ref):
    nr, nl = z_ref.shape
    out = de_ref[...] * (agg_ref[0] + agg_ref[1] + z_ref[...]) + b_ref[...]
    valid = lax.broadcasted_iota(jnp.int32, (nr, nl), 0) < 625
    validf = valid.astype(jnp.float32)
    gmax = jnp.max(jnp.where(valid, out, -jnp.inf))
    ex = jnp.exp(out - gmax) * validf
    sums16 = lax.dot_general(ex, r_ref[...], (((1,), (1,)), ((), ())),
                             preferred_element_type=jnp.float32)
    valid16 = lax.broadcasted_iota(jnp.int32, sums16.shape, 0) < 625
    lse_sum = jnp.sum(jnp.log(sums16 + (1.0 - valid16.astype(jnp.float32))))
    y_exp = jnp.dot(y_ref[...].astype(jnp.float32), r_ref[...],
                    preferred_element_type=jnp.float32)
    cls_f = (lax.broadcasted_iota(jnp.int32, (nr, nl), 1) % 8
             ).astype(jnp.float32)
    picked = jnp.sum(jnp.where(cls_f == y_exp, out, 0.0) * validf)
    n = 10000.0
    loss = (n * gmax + lse_sum - picked) / n
    out_ref[...] = jnp.full((1, 1), loss, jnp.float32)


def kernel(hiddens, edge_index, y, W, b):
    n, hid = hiddens.shape
    ncls = W.shape[1]
    e = edge_index.shape[1]
    npad = 10240
    nchw = 78
    emain = NW * nchw * CH
    ntail = (e - emain) // CH
    assert emain + ntail * CH == e and ncls == 8 and n == 10000

    edge3 = edge_index.reshape(2, e // CH, CH)
    ones_ch8 = jnp.ones((CH, ncls), jnp.float32)
    zeros_n8 = jnp.zeros((npad, ncls), jnp.float32)

    R = (jnp.eye(16, dtype=jnp.float32)[:, :, None]
         * jnp.ones((ncls,), jnp.float32)).reshape(16, 128)
    y16 = jnp.pad(y, (0, npad - n)).reshape(npad // 16, 16)
    b128 = jnp.tile(b, 16).reshape(1, 128)
    W2 = (jnp.eye(16, dtype=jnp.float32)[:, None, :, None]
          * W[None, :, None, :]).reshape(16 * hid, 128)
    h2 = jnp.pad(hiddens.reshape(n // 16, 16 * hid), ((0, 15), (0, 0)))

    deg2 = _make_deg_kernel(npad, ncls, nchw, ntail)(edge3, ones_ch8,
                                                     zeros_n8)

    xw = pl.pallas_call(
        _xw_body,
        out_shape=jax.ShapeDtypeStruct((npad // 16, 128), jnp.float32),
    )(h2, W2)

    z, dis_exp = pl.pallas_call(
        _z_body,
        out_shape=[
            jax.ShapeDtypeStruct((npad // 16, 128), jnp.float32),
            jax.ShapeDtypeStruct((npad // 16, 128), jnp.float32),
        ],
    )(xw, deg2.reshape(NC, npad // 16, 128))

    agg2 = _make_agg_kernel(npad, ncls, nchw, ntail)(
        edge3, z.reshape(npad, ncls), zeros_n8)

    loss = pl.pallas_call(
        _loss_body,
        out_shape=jax.ShapeDtypeStruct((1, 1), jnp.float32),
    )(agg2.reshape(NC, npad // 16, 128), z, dis_exp, y16, b128, R)

    return loss.reshape(())
